# baseline (speedup 1.0000x reference)
import functools

import jax
import jax.numpy as jnp
from jax import lax
from jax.experimental import pallas as pl
from jax.experimental.pallas import tpu as pltpu
from jax.experimental.pallas import tpu_sc as plsc

_B = 16384
_L = 20
_NW = 32
_ROWS_W = _B // _NW
_GROUP = 32
_NGROUP = _ROWS_W // _GROUP
_TOK = _GROUP * _L
_NCH = _TOK // 128
_CE = _ROWS_W * 16
_CCH = _CE // 128


def _sc_body(wid3, c1x, c2x, c3x, wemb, c1e, c2e, c3e,
             pooled_out, c1_out, c2_out, c3_out,
             idx_v, rows_v, out_v, cidx_v, crows_v, sem):
    w = lax.axis_index("c") * 16 + lax.axis_index("s")

    for ids_h, emb_h, out_h in ((c1x, c1e, c1_out),
                                (c2x, c2e, c2_out),
                                (c3x, c3e, c3_out)):
        pltpu.sync_copy(ids_h.at[w], cidx_v)

        def cbody(r, carry):
            cps = [pltpu.async_copy(emb_h.at[cidx_v.at[r * 16 + j]],
                                    crows_v.at[pl.ds((r * 16 + j) * 128, 128)],
                                    sem)
                   for j in range(16)]
            for cp in cps:
                cp.wait()
            return carry

        lax.fori_loop(0, _CCH // 16, cbody, 0)
        pltpu.sync_copy(crows_v, out_h.at[pl.ds(w * _CE, _CE)])

    def gbody(g, carry):
        pltpu.sync_copy(wid3.at[w * _NGROUP + g], idx_v)
        cps = [pltpu.async_copy(wemb.at[idx_v.at[j]],
                                rows_v.at[pl.ds(j * 128, 128)], sem)
               for j in range(_NCH)]
        for cp in cps:
            cp.wait()

        def rbody(r, c2):
            base = r * _L
            acc = [rows_v[base, pl.ds(f * 16, 16)] for f in range(4)]
            for l in range(1, _L):
                for f in range(4):
                    acc[f] = acc[f] + rows_v[base + l, pl.ds(f * 16, 16)]
            for f in range(4):
                out_v[r, pl.ds(f * 16, 16)] = acc[f]
            return c2

        lax.fori_loop(0, _GROUP, rbody, 0)
        pltpu.sync_copy(out_v,
                        pooled_out.at[pl.ds(w * _ROWS_W + g * _GROUP, _GROUP)])
        return carry

    lax.fori_loop(0, _NGROUP, gbody, 0)


_sc_gather = functools.partial(
    pl.kernel,
    out_type=[
        jax.ShapeDtypeStruct((_B, 64), jnp.float32),
        jax.ShapeDtypeStruct((_B * 16,), jnp.float32),
        jax.ShapeDtypeStruct((_B * 16,), jnp.float32),
        jax.ShapeDtypeStruct((_B * 16,), jnp.float32),
    ],
    mesh=plsc.VectorSubcoreMesh(core_axis_name="c", subcore_axis_name="s"),
    compiler_params=pltpu.CompilerParams(use_tc_tiling_on_sc=False),
    scratch_types=[
        pltpu.VMEM((_NCH, 128), jnp.int32),
        pltpu.VMEM((_TOK, 64), jnp.float32),
        pltpu.VMEM((_GROUP, 64), jnp.float32),
        pltpu.VMEM((_CCH, 128), jnp.int32),
        pltpu.VMEM((_CE,), jnp.float32),
        pltpu.SemaphoreType.DMA,
    ],
)(_sc_body)


def _ln(x, g, b, eps=1e-5):
    m = jnp.mean(x, axis=-1, keepdims=True)
    v = jnp.mean((x - m) ** 2, axis=-1, keepdims=True)
    return (x - m) / jnp.sqrt(v + eps) * g + b


def _gelu(x):
    return 0.5 * x * (1.0 + lax.erf(x * 0.7071067811865476))


_BLK = 1024


def _tc_body(wids_ref, pooled_ref, c1_ref, c2_ref, c3_ref, num_ref,
             tpw, tpb, tlg, tlb, npw, npb, nlg, nlb,
             f1w, f1b, flg, flb, f2w, f2b, out_ref):
    cnt = jnp.sum((wids_ref[...] != 0).astype(jnp.float32), axis=1,
                  keepdims=True)
    mean = pooled_ref[...] / jnp.maximum(cnt, 1.0)
    t = _gelu(_ln(jnp.dot(mean, tpw[...],
                          preferred_element_type=jnp.float32) + tpb[...],
                  tlg[...], tlb[...]))
    n = _gelu(_ln(jnp.dot(num_ref[...], npw[...],
                          preferred_element_type=jnp.float32) + npb[...],
                  nlg[...], nlb[...]))
    fused = jnp.concatenate([t, c1_ref[...], c2_ref[...], c3_ref[...], n],
                            axis=-1)
    h = _gelu(_ln(jnp.dot(fused, f1w[...],
                          preferred_element_type=jnp.float32) + f1b[...],
                  flg[...], flb[...]))
    out_ref[...] = jnp.dot(h, f2w[...],
                           preferred_element_type=jnp.float32) + f2b[...]


def _full(shape):
    return pl.BlockSpec(shape, lambda i: (0,) * len(shape))


_tc_encode = pl.pallas_call(
    _tc_body,
    grid=(_B // _BLK,),
    in_specs=[
        pl.BlockSpec((_BLK, _L), lambda i: (i, 0)),
        pl.BlockSpec((_BLK, 64), lambda i: (i, 0)),
        pl.BlockSpec((_BLK, 16), lambda i: (i, 0)),
        pl.BlockSpec((_BLK, 16), lambda i: (i, 0)),
        pl.BlockSpec((_BLK, 16), lambda i: (i, 0)),
        pl.BlockSpec((_BLK, 2), lambda i: (i, 0)),
        _full((64, 64)), _full((64,)), _full((64,)), _full((64,)),
        _full((2, 16)), _full((16,)), _full((16,)), _full((16,)),
        _full((128, 128)), _full((128,)), _full((128,)), _full((128,)),
        _full((128, 128)), _full((128,)),
    ],
    out_specs=pl.BlockSpec((_BLK, 128), lambda i: (i, 0)),
    out_shape=jax.ShapeDtypeStruct((_B, 128), jnp.float32),
)


def _elem_idx(ids):
    return (ids[:, None] * 16 +
            jnp.arange(16, dtype=jnp.int32)).reshape(_NW, _CCH, 128)


def kernel(word_ids, cat1_ids, cat2_ids, cat3_ids, numerics,
           word_emb, cat1_emb, cat2_emb, cat3_emb,
           t_proj_w, t_proj_b, t_ln_g, t_ln_b,
           n_proj_w, n_proj_b, n_ln_g, n_ln_b,
           f1_w, f1_b, f_ln_g, f_ln_b, f2_w, f2_b):
    wid3 = word_ids.reshape(_NW * _NGROUP, _NCH, 128)
    pooled, c1f, c2f, c3f = _sc_gather(
        wid3, _elem_idx(cat1_ids), _elem_idx(cat2_ids), _elem_idx(cat3_ids),
        word_emb, cat1_emb.reshape(-1), cat2_emb.reshape(-1),
        cat3_emb.reshape(-1))
    return _tc_encode(word_ids, pooled, c1f.reshape(_B, 16),
                      c2f.reshape(_B, 16), c3f.reshape(_B, 16), numerics,
                      t_proj_w.T, t_proj_b, t_ln_g, t_ln_b,
                      n_proj_w.T, n_proj_b, n_ln_g, n_ln_b,
                      f1_w.T, f1_b, f_ln_g, f_ln_b,
                      f2_w.T, f2_b)


# speedup vs baseline: 1.0830x; 1.0830x over previous
import functools

import jax
import jax.numpy as jnp
from jax import lax
from jax.experimental import pallas as pl
from jax.experimental.pallas import tpu as pltpu
from jax.experimental.pallas import tpu_sc as plsc

_B = 16384
_L = 20
_NW = 32
_ROWS_W = _B // _NW
_GW = 64
_NGW = _ROWS_W // _GW


def _sc_body(wid_t, c1i, c2i, c3i, wemb, c1e, c2e, c3e,
             pooled_out, recv_out, c1_out, c2_out, c3_out,
             idsv, recv, rowsbuf, out_v, cidsv, crows, sem):
    w = lax.axis_index("c") * 16 + lax.axis_index("s")
    base = w * _ROWS_W

    pltpu.sync_copy(wid_t.at[:, pl.ds(base, _ROWS_W)], idsv)

    def kbody(j, carry):
        acc = jnp.zeros((16,), jnp.float32)
        for l in range(_L):
            acc = acc + jnp.where(idsv[l, pl.ds(j * 16, 16)] != 0, 1.0, 0.0)
        recv[pl.ds(j * 16, 16)] = jnp.maximum(acc, 1.0)
        return carry

    lax.fori_loop(0, _ROWS_W // 16, kbody, 0)

    for ids_h, emb_h, out_h in ((c1i, c1e, c1_out),
                                (c2i, c2e, c2_out),
                                (c3i, c3e, c3_out)):
        pltpu.sync_copy(ids_h.at[pl.ds(base, _ROWS_W)], cidsv)
        cps = [pltpu.async_copy(emb_h.at[cidsv.at[pl.ds(j * 128, 128)]],
                                crows.at[pl.ds(j * 128, 128)], sem)
               for j in range(_ROWS_W // 128)]
        for cp in cps:
            cp.wait()
        pltpu.sync_copy(crows, out_h.at[pl.ds(base, _ROWS_W)])

    def gbody(g, carry):
        cps = [pltpu.async_copy(wemb.at[idsv.at[l, pl.ds(g * _GW, _GW)]],
                                rowsbuf.at[l], sem)
               for l in range(_L)]
        for cp in cps:
            cp.wait()

        def rbody(r, c2):
            for f in range(4):
                acc = rowsbuf[0, r, pl.ds(f * 16, 16)]
                for l in range(1, _L):
                    acc = acc + rowsbuf[l, r, pl.ds(f * 16, 16)]
                out_v[r, pl.ds(f * 16, 16)] = acc
            return c2

        lax.fori_loop(0, _GW, rbody, 0)
        pltpu.sync_copy(out_v, pooled_out.at[pl.ds(base + g * _GW, _GW)])
        return carry

    lax.fori_loop(0, _NGW, gbody, 0)
    pltpu.sync_copy(recv, recv_out.at[pl.ds(base, _ROWS_W)])


_sc_gather = functools.partial(
    pl.kernel,
    out_type=[
        jax.ShapeDtypeStruct((_B, 64), jnp.float32),
        jax.ShapeDtypeStruct((_B,), jnp.float32),
        jax.ShapeDtypeStruct((_B, 16), jnp.float32),
        jax.ShapeDtypeStruct((_B, 16), jnp.float32),
        jax.ShapeDtypeStruct((_B, 16), jnp.float32),
    ],
    mesh=plsc.VectorSubcoreMesh(core_axis_name="c", subcore_axis_name="s"),
    compiler_params=pltpu.CompilerParams(use_tc_tiling_on_sc=False),
    scratch_types=[
        pltpu.VMEM((_L, _ROWS_W), jnp.int32),
        pltpu.VMEM((_ROWS_W,), jnp.float32),
        pltpu.VMEM((_L, _GW, 64), jnp.float32),
        pltpu.VMEM((_GW, 64), jnp.float32),
        pltpu.VMEM((_ROWS_W,), jnp.int32),
        pltpu.VMEM((_ROWS_W, 16), jnp.float32),
        pltpu.SemaphoreType.DMA,
    ],
)(_sc_body)


def _ln(x, g, b, eps=1e-5):
    m = jnp.mean(x, axis=-1, keepdims=True)
    v = jnp.mean((x - m) ** 2, axis=-1, keepdims=True)
    return (x - m) / jnp.sqrt(v + eps) * g + b


def _gelu(x):
    return 0.5 * x * (1.0 + lax.erf(x * 0.7071067811865476))


_BLK = 1024


def _tc_body(pooled_ref, rec_ref, c1_ref, c2_ref, c3_ref, numt_ref,
             tpw, tpb, tlg, tlb, npw, npb, nlg, nlb,
             f1w, f1b, flg, flb, f2w, f2b, out_ref):
    rcol = lax.dot_general(rec_ref[...], jnp.ones((1, 1), jnp.float32),
                           (((0,), (0,)), ((), ())),
                           preferred_element_type=jnp.float32)
    mean = pooled_ref[...] / rcol
    t = _gelu(_ln(jnp.dot(mean, tpw[...],
                          preferred_element_type=jnp.float32) + tpb[...],
                  tlg[...], tlb[...]))
    nlin = lax.dot_general(numt_ref[...], npw[...], (((0,), (0,)), ((), ())),
                           preferred_element_type=jnp.float32)
    n = _gelu(_ln(nlin + npb[...], nlg[...], nlb[...]))
    fused = jnp.concatenate([t, c1_ref[...], c2_ref[...], c3_ref[...], n],
                            axis=-1)
    h = _gelu(_ln(jnp.dot(fused, f1w[...],
                          preferred_element_type=jnp.float32) + f1b[...],
                  flg[...], flb[...]))
    out_ref[...] = jnp.dot(h, f2w[...],
                           preferred_element_type=jnp.float32) + f2b[...]


def _full(shape):
    return pl.BlockSpec(shape, lambda i: (0,) * len(shape))


_tc_encode = pl.pallas_call(
    _tc_body,
    grid=(_B // _BLK,),
    in_specs=[
        pl.BlockSpec((_BLK, 64), lambda i: (i, 0)),
        pl.BlockSpec((1, _BLK), lambda i: (0, i)),
        pl.BlockSpec((_BLK, 16), lambda i: (i, 0)),
        pl.BlockSpec((_BLK, 16), lambda i: (i, 0)),
        pl.BlockSpec((_BLK, 16), lambda i: (i, 0)),
        pl.BlockSpec((2, _BLK), lambda i: (0, i)),
        _full((64, 64)), _full((64,)), _full((64,)), _full((64,)),
        _full((2, 16)), _full((16,)), _full((16,)), _full((16,)),
        _full((128, 128)), _full((128,)), _full((128,)), _full((128,)),
        _full((128, 128)), _full((128,)),
    ],
    out_specs=pl.BlockSpec((_BLK, 128), lambda i: (i, 0)),
    out_shape=jax.ShapeDtypeStruct((_B, 128), jnp.float32),
)


def kernel(word_ids, cat1_ids, cat2_ids, cat3_ids, numerics,
           word_emb, cat1_emb, cat2_emb, cat3_emb,
           t_proj_w, t_proj_b, t_ln_g, t_ln_b,
           n_proj_w, n_proj_b, n_ln_g, n_ln_b,
           f1_w, f1_b, f_ln_g, f_ln_b, f2_w, f2_b):
    pooled, rec, c1, c2, c3 = _sc_gather(
        word_ids.T, cat1_ids, cat2_ids, cat3_ids,
        word_emb, cat1_emb, cat2_emb, cat3_emb)
    return _tc_encode(pooled, rec.reshape(1, _B), c1, c2, c3, numerics.T,
                      t_proj_w.T, t_proj_b, t_ln_g, t_ln_b,
                      n_proj_w.T, n_proj_b, n_ln_g, n_ln_b,
                      f1_w.T, f1_b, f_ln_g, f_ln_b,
                      f2_w.T, f2_b)


# speedup vs baseline: 1.1068x; 1.0220x over previous
import functools

import jax
import jax.numpy as jnp
from jax import lax
from jax.experimental import pallas as pl
from jax.experimental.pallas import tpu as pltpu
from jax.experimental.pallas import tpu_sc as plsc

_B = 16384
_L = 20
_NW = 32
_ROWS_W = _B // _NW
_GW = 64
_NGW = _ROWS_W // _GW


def _detile_body(x_ref, o_ref):
    o_ref[...] = x_ref[...].reshape(_L * _B // 128, 128)


_detile = pl.pallas_call(
    _detile_body,
    in_specs=[pl.BlockSpec((_L, _B), lambda: (0, 0))],
    out_specs=pl.BlockSpec((_L * _B // 128, 128), lambda: (0, 0)),
    out_shape=jax.ShapeDtypeStruct((_L * _B // 128, 128), jnp.int32),
)


def _sc_body(wid_lin, c1i, c2i, c3i, wemb, c1e, c2e, c3e,
             pack_out, cnt_out,
             idsv, cntv, rowsbuf, out_v, cidsv, crows, sem):
    w = lax.axis_index("c") * 16 + lax.axis_index("s")
    base = w * _ROWS_W

    for l in range(_L):
        pltpu.sync_copy(wid_lin.at[pl.ds(l * 128 + w * 4, 4)], idsv.at[l])

    def kbody(j, carry):
        acc = jnp.zeros((16,), jnp.float32)
        for l in range(_L):
            acc = acc + jnp.where(
                idsv[l, j // 8, pl.ds((j % 8) * 16, 16)] != 0, 1.0, 0.0)
        cntv[pl.ds(j * 16, 16)] = jnp.maximum(acc, 1.0)
        return carry

    lax.fori_loop(0, _ROWS_W // 16, kbody, 0)
    pltpu.sync_copy(cntv, cnt_out.at[pl.ds(base, _ROWS_W)])

    for off, ids_h, emb_h in ((64, c1i, c1e), (80, c2i, c2e), (96, c3i, c3e)):
        pltpu.sync_copy(ids_h.at[pl.ds(base, _ROWS_W)], cidsv)
        cps = [pltpu.async_copy(emb_h.at[cidsv.at[pl.ds(j * 128, 128)]],
                                crows.at[pl.ds(j * 128, 128)], sem)
               for j in range(_ROWS_W // 128)]
        for cp in cps:
            cp.wait()
        pltpu.sync_copy(crows,
                        pack_out.at[pl.ds(base, _ROWS_W), pl.ds(off, 16)])

    def gbody(g, carry):
        cps = [pltpu.async_copy(
                   wemb.at[idsv.at[l, g // 2, pl.ds((g % 2) * 64, 64)]],
                   rowsbuf.at[l], sem)
               for l in range(_L)]
        for cp in cps:
            cp.wait()

        def rbody(r, c2):
            for f in range(4):
                acc = rowsbuf[0, r, pl.ds(f * 16, 16)]
                for l in range(1, _L):
                    acc = acc + rowsbuf[l, r, pl.ds(f * 16, 16)]
                out_v[r, pl.ds(f * 16, 16)] = acc
            return c2

        lax.fori_loop(0, _GW, rbody, 0)
        pltpu.sync_copy(out_v,
                        pack_out.at[pl.ds(base + g * _GW, _GW), pl.ds(0, 64)])
        return carry

    lax.fori_loop(0, _NGW, gbody, 0)


_sc_gather = functools.partial(
    pl.kernel,
    out_type=[
        jax.ShapeDtypeStruct((_B, 128), jnp.float32),
        jax.ShapeDtypeStruct((_B,), jnp.float32),
    ],
    mesh=plsc.VectorSubcoreMesh(core_axis_name="c", subcore_axis_name="s"),
    compiler_params=pltpu.CompilerParams(use_tc_tiling_on_sc=False),
    scratch_types=[
        pltpu.VMEM((_L, 4, 128), jnp.int32),
        pltpu.VMEM((_ROWS_W,), jnp.float32),
        pltpu.VMEM((_L, _GW, 64), jnp.float32),
        pltpu.VMEM((_GW, 64), jnp.float32),
        pltpu.VMEM((_ROWS_W,), jnp.int32),
        pltpu.VMEM((_ROWS_W, 16), jnp.float32),
        pltpu.SemaphoreType.DMA,
    ],
)(_sc_body)


def _ln(x, g, b, eps=1e-5):
    m = jnp.mean(x, axis=-1, keepdims=True)
    v = jnp.mean((x - m) ** 2, axis=-1, keepdims=True)
    return (x - m) / jnp.sqrt(v + eps) * g + b


def _gelu(x):
    return 0.5 * x * (1.0 + lax.erf(x * 0.7071067811865476))


_BLK = 1024


def _tc_body(pack_ref, cnt_ref, numt_ref,
             tpw, tpb, tlg, tlb, npw, npb, nlg, nlb,
             f1w, f1b, flg, flb, f2w, f2b, out_ref):
    pack = pack_ref[...]
    ccol = lax.dot_general(cnt_ref[...], jnp.ones((1, 1), jnp.float32),
                           (((0,), (0,)), ((), ())),
                           preferred_element_type=jnp.float32)
    mean = pack[:, 0:64] / ccol
    t = _gelu(_ln(jnp.dot(mean, tpw[...],
                          preferred_element_type=jnp.float32) + tpb[...],
                  tlg[...], tlb[...]))
    nlin = lax.dot_general(numt_ref[...], npw[...], (((0,), (0,)), ((), ())),
                           preferred_element_type=jnp.float32)
    n = _gelu(_ln(nlin + npb[...], nlg[...], nlb[...]))
    fused = jnp.concatenate([t, pack[:, 64:112], n], axis=-1)
    h = _gelu(_ln(jnp.dot(fused, f1w[...],
                          preferred_element_type=jnp.float32) + f1b[...],
                  flg[...], flb[...]))
    out_ref[...] = jnp.dot(h, f2w[...],
                           preferred_element_type=jnp.float32) + f2b[...]


def _full(shape):
    return pl.BlockSpec(shape, lambda i: (0,) * len(shape))


_tc_encode = pl.pallas_call(
    _tc_body,
    grid=(_B // _BLK,),
    in_specs=[
        pl.BlockSpec((_BLK, 128), lambda i: (i, 0)),
        pl.BlockSpec((1, _BLK), lambda i: (0, i)),
        pl.BlockSpec((2, _BLK), lambda i: (0, i)),
        _full((64, 64)), _full((64,)), _full((64,)), _full((64,)),
        _full((2, 16)), _full((16,)), _full((16,)), _full((16,)),
        _full((128, 128)), _full((128,)), _full((128,)), _full((128,)),
        _full((128, 128)), _full((128,)),
    ],
    out_specs=pl.BlockSpec((_BLK, 128), lambda i: (i, 0)),
    out_shape=jax.ShapeDtypeStruct((_B, 128), jnp.float32),
)


def kernel(word_ids, cat1_ids, cat2_ids, cat3_ids, numerics,
           word_emb, cat1_emb, cat2_emb, cat3_emb,
           t_proj_w, t_proj_b, t_ln_g, t_ln_b,
           n_proj_w, n_proj_b, n_ln_g, n_ln_b,
           f1_w, f1_b, f_ln_g, f_ln_b, f2_w, f2_b):
    wid_lin = _detile(word_ids.T)
    pack, cnt = _sc_gather(wid_lin, cat1_ids, cat2_ids, cat3_ids,
                           word_emb, cat1_emb, cat2_emb, cat3_emb)
    return _tc_encode(pack, cnt.reshape(1, _B), numerics.T,
                      t_proj_w.T, t_proj_b, t_ln_g, t_ln_b,
                      n_proj_w.T, n_proj_b, n_ln_g, n_ln_b,
                      f1_w.T, f1_b, f_ln_g, f_ln_b,
                      f2_w.T, f2_b)
